# baseline (speedup 1.0000x reference)
import functools

import jax
import jax.numpy as jnp
from jax import lax
from jax.experimental import pallas as pl
from jax.experimental.pallas import tpu as pltpu
from jax.experimental.pallas import tpu_sc as plsc

B, H, W, C = 16, 384, 384, 3
N = B * H * W
NTILES = 32
PER_TILE = N // NTILES
SUB = 3072
NSUB = PER_TILE // SUB
UNROLL = 4
RPR = W // 12
ROWS_B = H * RPR
SC_ROWS = 8 * ROWS_B
HIMASK = jnp.int32(-65536)


def _sampler_body(cx_hbm, cy_hbm, tab_hbm, out_hbm, cbx, cby,
                  ib0, ib1, gr0, gr1, obuf, shared, sem):
    cix = lax.axis_index("c")
    six = lax.axis_index("s")
    wid = cix * 16 + six
    ebase = wid * PER_TILE
    rowbase = (six // 2) * ROWS_B
    iota = lax.iota(jnp.int32, 16)

    strip = SC_ROWS // 16
    pltpu.async_copy(
        tab_hbm.at[pl.ds(cix * SC_ROWS + six * strip, strip)],
        shared.at[pl.ds(six * strip, strip)], sem).wait()
    plsc.subcore_barrier()

    def subchunk(s, carry):
        e0 = pl.multiple_of(ebase + s * SUB, SUB)
        pltpu.sync_copy(cx_hbm.at[pl.ds(e0, SUB)], cbx)
        pltpu.sync_copy(cy_hbm.at[pl.ds(e0, SUB)], cby)

        @plsc.parallel_loop(0, SUB, step=16, unroll=UNROLL)
        def compute(i):
            if True:
                c = pl.multiple_of(i, 16)
                xv = cbx[pl.ds(c, 16)]
                yv = cby[pl.ds(c, 16)]
                x0 = xv.astype(jnp.int32)
                y0 = yv.astype(jnp.int32)
                x0c = jnp.minimum(x0, W - 1)
                xd12 = (x0c * 683) >> 13
                y0c = jnp.minimum(y0, H - 1)
                y1c = jnp.minimum(y0 + 1, H - 1)
                ib0[pl.ds(c, 16)] = rowbase + y0c * RPR + xd12
                ib1[pl.ds(c, 16)] = rowbase + y1c * RPR + xd12

        pltpu.async_copy(shared.at[ib0], gr0, sem)
        pltpu.async_copy(shared.at[ib1], gr1, sem)
        pltpu.make_async_copy(shared.at[ib0], gr0, sem).wait()
        pltpu.make_async_copy(shared.at[ib1], gr1, sem).wait()

        @plsc.parallel_loop(0, SUB, step=16, unroll=UNROLL)
        def combine(i):
            if True:
                c = pl.multiple_of(i, 16)
                ridx = c + iota
                xv = cbx[pl.ds(c, 16)]
                yv = cby[pl.ds(c, 16)]
                x0 = xv.astype(jnp.int32)
                y0 = yv.astype(jnp.int32)
                fx = xv - x0.astype(jnp.float32)
                fy = yv - y0.astype(jnp.float32)
                x0c = jnp.minimum(x0, W - 1)
                xd12 = (x0c * 683) >> 13
                o = x0c - xd12 * 12
                wl = o >> 1
                wr = (o + 1) >> 1
                a0 = plsc.load_gather(gr0, [ridx, wl])
                b0 = plsc.load_gather(gr0, [ridx, wr])
                a1 = plsc.load_gather(gr1, [ridx, wl])
                b1 = plsc.load_gather(gr1, [ridx, wr])
                pL = (o & 1) == 1
                v00 = plsc.bitcast(jnp.where(pL, a0 & HIMASK, a0 << 16), jnp.float32)
                v10 = plsc.bitcast(jnp.where(pL, b0 << 16, b0 & HIMASK), jnp.float32)
                v01 = plsc.bitcast(jnp.where(pL, a1 & HIMASK, a1 << 16), jnp.float32)
                v11 = plsc.bitcast(jnp.where(pL, b1 << 16, b1 & HIMASK), jnp.float32)
                wx0 = 1.0 - fx
                wy0 = 1.0 - fy
                res = (wx0 * wy0) * v00 + (wx0 * fy) * v01
                res = res + ((fx * wy0) * v10 + (fx * fy) * v11)
                obuf[pl.ds(c, 16)] = res
        pltpu.sync_copy(obuf, out_hbm.at[pl.ds(e0, SUB)])
        return carry

    lax.fori_loop(0, NSUB, subchunk, 0)


def kernel(imgs, coords):
    src16 = imgs.reshape(-1)[:N].astype(jnp.bfloat16)
    rows = N // 12
    vpad = jnp.concatenate([src16, jnp.zeros((16,), jnp.bfloat16)])
    m1 = src16.reshape(rows, 12)
    m2 = lax.dynamic_slice_in_dim(vpad, 12, N).reshape(rows, 12)[:, :4]
    tab16 = jnp.concatenate([m1, m2], axis=1)
    tabw = lax.bitcast_convert_type(tab16.reshape(rows, 8, 2), jnp.int32)
    cpl = coords.reshape(N, 2)
    cx = cpl[:, 0]
    cy = cpl[:, 1]
    mesh = plsc.VectorSubcoreMesh(core_axis_name="c", subcore_axis_name="s")
    run = functools.partial(
        pl.kernel,
        mesh=mesh,
        compiler_params=pltpu.CompilerParams(
            needs_layout_passes=False, use_tc_tiling_on_sc=False),
        out_type=jax.ShapeDtypeStruct((N,), jnp.float32),
        scratch_types=[
            pltpu.VMEM((SUB,), jnp.float32),
            pltpu.VMEM((SUB,), jnp.float32),
            pltpu.VMEM((SUB,), jnp.int32),
            pltpu.VMEM((SUB,), jnp.int32),
            pltpu.VMEM((SUB, 8), jnp.int32),
            pltpu.VMEM((SUB, 8), jnp.int32),
            pltpu.VMEM((SUB,), jnp.float32),
            pltpu.VMEM_SHARED((SC_ROWS, 8), jnp.int32),
            pltpu.SemaphoreType.DMA,
        ],
    )(_sampler_body)
    out = run(cx, cy, tabw)
    return out.reshape(B, H, W, 1)
